# baseline (speedup 1.0000x reference)
import math

import jax
import jax.numpy as jnp
from jax import lax
from jax.experimental import pallas as pl
from jax.experimental.pallas import tpu as pltpu
from jax.experimental.pallas import tpu_sc as plsc

ALPHA_INIT = 0.01
ACT_SHIFT = math.log(1.0 / (1.0 - ALPHA_INIT) - 1.0)
TOTAL = 2097152
NRAYS = 16384
NTILES = 32
KREG = 32
NREG = NTILES * KREG
RLEN = TOTAL // NREG
BLK = 512
NBLK = RLEN // BLK

_MESH = plsc.VectorSubcoreMesh(core_axis_name="c", subcore_axis_name="s")
_CPARAMS = pltpu.CompilerParams(use_tc_tiling_on_sc=False,
                                needs_layout_passes=False)


def _wid():
    return lax.axis_index("s") * 2 + lax.axis_index("c")


def _splat(x):
    return jnp.full((16,), x, jnp.int32)


def _elementwise(xg):
    e = jnp.exp(xg + jnp.float32(ACT_SHIFT))
    inv = 1.0 / (1.0 + e)
    q = jnp.maximum(inv, jnp.float32(1e-10))
    alpha = 1.0 - inv
    return alpha, q


def _init_prev_rid(rid_hbm, idx_v, prev_v, sem, base_reg, iota):
    idx_v[pl.ds(0, 16)] = jnp.maximum((base_reg + iota) * RLEN - 1, 0)
    idx_v[pl.ds(16, 16)] = (base_reg + 16 + iota) * RLEN - 1
    pltpu.async_copy(rid_hbm.at[idx_v], prev_v, sem).wait()
    pa = jnp.where((base_reg + iota) == 0, -1, prev_v[pl.ds(0, 16)])
    pb = prev_v[pl.ds(16, 16)]
    return pa, pb


def _issue_in(dens_hbm, rid_hbm, dbuf, rbuf, dsem, rsem, base_reg, b):
    cps = []
    for k in range(KREG):
        src = pl.ds((base_reg + k) * RLEN + b * BLK, BLK)
        dst = pl.ds(k * BLK, BLK)
        cps.append(pltpu.async_copy(dens_hbm.at[src], dbuf.at[dst], dsem))
        cps.append(pltpu.async_copy(rid_hbm.at[src], rbuf.at[dst], rsem))
    return cps


def _compose(cur, tail, has):
    return tail * jnp.where(has != 0, jnp.float32(1.0), cur)


def _k1_body(dens_hbm, rid_hbm, tails_hbm, has_hbm, ttail_hbm, thas_hbm,
             d0, d1, r0, r1, idx_v, prev_v, stfa, stfb, stia, stib, stt, sth,
             sp, sd0, sd1, sr0, sr1):
    w = _wid()
    base_reg = w * KREG
    iota = lax.iota(jnp.int32, 16)
    rowa = iota * BLK
    rowb = (16 + iota) * BLK
    preva, prevb = _init_prev_rid(rid_hbm, idx_v, prev_v, sp, base_reg, iota)

    dbuf, rbuf = (d0, d1), (r0, r1)
    dsem, rsem = (sd0, sd1), (sr0, sr1)

    def issue(b):
        return _issue_in(dens_hbm, rid_hbm, dbuf[b % 2], rbuf[b % 2],
                         dsem[b % 2], rsem[b % 2], base_reg, b)

    pend = issue(0)
    ones = jnp.ones((16,), jnp.float32)
    zeros = jnp.zeros((16,), jnp.int32)
    carry = (ones, zeros, preva, ones, zeros, prevb)

    for b in range(NBLK):
        nxt = issue(b + 1) if b + 1 < NBLK else None
        for cp in pend:
            cp.wait()
        db = dbuf[b % 2]
        rb = rbuf[b % 2]

        @plsc.parallel_loop(0, BLK, carry=carry, unroll=8)
        def _sweep1(j, c):
            ta, ha, pa, tb, hb, pb = c
            col = _splat(j)
            xa = plsc.load_gather(db, [rowa + col])
            ra = plsc.load_gather(rb, [rowa + col])
            xb = plsc.load_gather(db, [rowb + col])
            rb_ = plsc.load_gather(rb, [rowb + col])
            bnda = ra != pa
            bndb = rb_ != pb
            _, qa = _elementwise(xa)
            _, qb = _elementwise(xb)
            ta = jnp.where(bnda, qa, ta * qa)
            tb = jnp.where(bndb, qb, tb * qb)
            ha = jnp.where(bnda, 1, ha)
            hb = jnp.where(bndb, 1, hb)
            return ta, ha, ra, tb, hb, rb_

        carry = _sweep1
        pend = nxt

    ta, ha, _, tb, hb, _ = carry
    stfa[...] = ta
    stfb[...] = tb
    stia[...] = ha
    stib[...] = hb
    pltpu.sync_copy(stfa, tails_hbm.at[pl.ds(base_reg, 16)])
    pltpu.sync_copy(stfb, tails_hbm.at[pl.ds(base_reg + 16, 16)])
    pltpu.sync_copy(stia, has_hbm.at[pl.ds(base_reg, 16)])
    pltpu.sync_copy(stib, has_hbm.at[pl.ds(base_reg + 16, 16)])

    cur = jnp.ones((16,), jnp.float32)
    anyh = jnp.zeros((16,), jnp.int32)
    for k in range(KREG):
        tv, hv = (stfa, stia) if k < 16 else (stfb, stib)
        ti = plsc.load_gather(tv, [_splat(k % 16)])
        hi = plsc.load_gather(hv, [_splat(k % 16)])
        cur = _compose(cur, ti, hi)
        anyh = jnp.where(hi != 0, 1, anyh)
    stt[...] = cur
    sth[...] = anyh
    pltpu.sync_copy(stt, ttail_hbm.at[pl.ds(w * 16, 16)])
    pltpu.sync_copy(sth, thas_hbm.at[pl.ds(w * 16, 16)])


def _k2_body(dens_hbm, rid_hbm, tails_hbm, has_hbm, ttail_hbm, thas_hbm,
             w_hbm, slabs_hbm,
             d0, d1, r0, r1, w0, w1, seg_v, mt_v, mh_v, tt_v, th_v,
             idx_v, prev_v,
             sp, sd0, sd1, sr0, sr1, sw0, sw1):
    w = _wid()
    base_reg = w * KREG
    iota = lax.iota(jnp.int32, 16)
    rowa = iota * BLK
    rowb = (16 + iota) * BLK
    preva, prevb = _init_prev_rid(rid_hbm, idx_v, prev_v, sp, base_reg, iota)

    dbuf, rbuf, wbuf = (d0, d1), (r0, r1), (w0, w1)
    dsem, rsem, wsem = (sd0, sd1), (sr0, sr1), (sw0, sw1)

    def issue(b):
        return _issue_in(dens_hbm, rid_hbm, dbuf[b % 2], rbuf[b % 2],
                         dsem[b % 2], rsem[b % 2], base_reg, b)

    pend = issue(0)

    pltpu.sync_copy(ttail_hbm, tt_v)
    pltpu.sync_copy(thas_hbm, th_v)
    pltpu.sync_copy(tails_hbm.at[pl.ds(base_reg, KREG)], mt_v)
    pltpu.sync_copy(has_hbm.at[pl.ds(base_reg, KREG)], mh_v)

    def cstep(i, cur):
        ti = plsc.load_gather(tt_v, [_splat(0) + i * 16])
        hi = plsc.load_gather(th_v, [_splat(0) + i * 16])
        return _compose(cur, ti, hi)

    cur = lax.fori_loop(0, w, cstep, jnp.ones((16,), jnp.float32))

    def estep(k, c):
        cur, ta, tb = c
        ta = jnp.where(iota == k, cur, ta)
        tb = jnp.where(iota == (k - 16), cur, tb)
        ti = plsc.load_gather(mt_v, [_splat(0) + k])
        hi = plsc.load_gather(mh_v, [_splat(0) + k])
        return _compose(cur, ti, hi), ta, tb

    ones16 = jnp.ones((16,), jnp.float32)
    _, ta, tb = lax.fori_loop(0, KREG, estep, (cur, ones16, ones16))

    nanv = jnp.full((16,), jnp.nan, jnp.float32)

    @plsc.parallel_loop(0, NRAYS // 16, unroll=8)
    def _init(i):
        seg_v[pl.ds(pl.multiple_of(i * 16, 16), 16)] = nanv

    carry = (ta, preva, tb, prevb)
    wpend = [None, None]
    for b in range(NBLK):
        nxt = issue(b + 1) if b + 1 < NBLK else None
        for cp in pend:
            cp.wait()
        db = dbuf[b % 2]
        rb = rbuf[b % 2]
        wb = wbuf[b % 2]
        if wpend[b % 2] is not None:
            for cp in wpend[b % 2]:
                cp.wait()

        @plsc.parallel_loop(0, BLK, carry=carry, unroll=8)
        def _sweep2(j, c):
            ta, pa, tb, pb = c
            col = _splat(j)
            xa = plsc.load_gather(db, [rowa + col])
            ra = plsc.load_gather(rb, [rowa + col])
            xb = plsc.load_gather(db, [rowb + col])
            rb_ = plsc.load_gather(rb, [rowb + col])
            bnda = ra != pa
            bndb = rb_ != pb
            plsc.store_scatter(seg_v, [pa], ta, mask=bnda & (pa >= 0))
            plsc.store_scatter(seg_v, [pb], tb, mask=bndb)
            ta = jnp.where(bnda, jnp.float32(1.0), ta)
            tb = jnp.where(bndb, jnp.float32(1.0), tb)
            alpha_a, qa = _elementwise(xa)
            alpha_b, qb = _elementwise(xb)
            plsc.store_scatter(wb, [rowa + col], alpha_a * ta)
            plsc.store_scatter(wb, [rowb + col], alpha_b * tb)
            ta = ta * qa
            tb = tb * qb
            return ta, ra, tb, rb_

        carry = _sweep2
        cws = []
        for k in range(KREG):
            dst = pl.ds((base_reg + k) * RLEN + b * BLK, BLK)
            cws.append(pltpu.async_copy(
                wb.at[pl.ds(k * BLK, BLK)], w_hbm.at[dst], wsem[b % 2]))
        wpend[b % 2] = cws
        pend = nxt

    ta, pa, tb, pb = carry
    last = (base_reg + 16 + iota) == (NREG - 1)
    plsc.store_scatter(seg_v, [pb], tb, mask=last)

    for cws in wpend:
        if cws is not None:
            for cp in cws:
                cp.wait()
    pltpu.sync_copy(seg_v, slabs_hbm.at[pl.ds(w * NRAYS, NRAYS)])


def _k3_body(slabs_hbm, ainv_hbm, all_v, out_v, sem):
    w = _wid()
    nper = NRAYS // NTILES
    base_ray = w * nper
    cps = [pltpu.async_copy(slabs_hbm.at[pl.ds(tt * NRAYS + base_ray, nper)],
                            all_v.at[pl.ds(tt * nper, nper)], sem)
           for tt in range(NTILES)]
    for cp in cps:
        cp.wait()

    @plsc.parallel_loop(0, nper // 16, unroll=2)
    def _fold(i):
        off = pl.multiple_of(i * 16, 16)
        a = all_v[pl.ds(off, 16)]
        for tt in range(1, NTILES):
            b = all_v[pl.ds(tt * nper + off, 16)]
            a = jnp.where(a != a, b, a)
        out_v[pl.ds(off, 16)] = jnp.where(a != a, jnp.float32(1.0), a)

    pltpu.sync_copy(out_v, ainv_hbm.at[pl.ds(base_ray, nper)])


_k1 = pl.kernel(
    _k1_body,
    out_type=(jax.ShapeDtypeStruct((NREG,), jnp.float32),
              jax.ShapeDtypeStruct((NREG,), jnp.int32),
              jax.ShapeDtypeStruct((NTILES * 16,), jnp.float32),
              jax.ShapeDtypeStruct((NTILES * 16,), jnp.int32)),
    mesh=_MESH,
    compiler_params=_CPARAMS,
    scratch_types=[
        pltpu.VMEM((KREG * BLK,), jnp.float32), pltpu.VMEM((KREG * BLK,), jnp.float32),
        pltpu.VMEM((KREG * BLK,), jnp.int32), pltpu.VMEM((KREG * BLK,), jnp.int32),
        pltpu.VMEM((32,), jnp.int32), pltpu.VMEM((32,), jnp.int32),
        pltpu.VMEM((16,), jnp.float32), pltpu.VMEM((16,), jnp.float32),
        pltpu.VMEM((16,), jnp.int32), pltpu.VMEM((16,), jnp.int32),
        pltpu.VMEM((16,), jnp.float32), pltpu.VMEM((16,), jnp.int32),
        pltpu.SemaphoreType.DMA, pltpu.SemaphoreType.DMA, pltpu.SemaphoreType.DMA,
        pltpu.SemaphoreType.DMA, pltpu.SemaphoreType.DMA,
    ],
)

_k2 = pl.kernel(
    _k2_body,
    out_type=(jax.ShapeDtypeStruct((TOTAL,), jnp.float32),
              jax.ShapeDtypeStruct((NTILES * NRAYS,), jnp.float32)),
    mesh=_MESH,
    compiler_params=_CPARAMS,
    scratch_types=[
        pltpu.VMEM((KREG * BLK,), jnp.float32), pltpu.VMEM((KREG * BLK,), jnp.float32),
        pltpu.VMEM((KREG * BLK,), jnp.int32), pltpu.VMEM((KREG * BLK,), jnp.int32),
        pltpu.VMEM((KREG * BLK,), jnp.float32), pltpu.VMEM((KREG * BLK,), jnp.float32),
        pltpu.VMEM((NRAYS,), jnp.float32),
        pltpu.VMEM((KREG,), jnp.float32), pltpu.VMEM((KREG,), jnp.int32),
        pltpu.VMEM((NTILES * 16,), jnp.float32), pltpu.VMEM((NTILES * 16,), jnp.int32),
        pltpu.VMEM((32,), jnp.int32), pltpu.VMEM((32,), jnp.int32),
        pltpu.SemaphoreType.DMA, pltpu.SemaphoreType.DMA, pltpu.SemaphoreType.DMA,
        pltpu.SemaphoreType.DMA, pltpu.SemaphoreType.DMA, pltpu.SemaphoreType.DMA,
        pltpu.SemaphoreType.DMA,
    ],
)

_k3 = pl.kernel(
    _k3_body,
    out_type=jax.ShapeDtypeStruct((NRAYS,), jnp.float32),
    mesh=_MESH,
    compiler_params=_CPARAMS,
    scratch_types=[
        pltpu.VMEM((NRAYS,), jnp.float32),
        pltpu.VMEM((NRAYS // NTILES,), jnp.float32),
        pltpu.SemaphoreType.DMA,
    ],
)


def kernel(density, ray_id, N):
    del N
    tails, has, ttail, thas = _k1(density, ray_id)
    weights, slabs = _k2(density, ray_id, tails, has, ttail, thas)
    alphainv = _k3(slabs)
    return weights, alphainv


# speedup vs baseline: 1.1395x; 1.1395x over previous
import math

import jax
import jax.numpy as jnp
from jax import lax
from jax.experimental import pallas as pl
from jax.experimental.pallas import tpu as pltpu
from jax.experimental.pallas import tpu_sc as plsc

ALPHA_INIT = 0.01
ACT_SHIFT = math.log(1.0 / (1.0 - ALPHA_INIT) - 1.0)
TOTAL = 2097152
NRAYS = 16384
NTILES = 32
KREG = 32
NREG = NTILES * KREG
RLEN = TOTAL // NREG
BLK = 512
NBLK = RLEN // BLK

_MESH = plsc.VectorSubcoreMesh(core_axis_name="c", subcore_axis_name="s")
_CPARAMS = pltpu.CompilerParams(use_tc_tiling_on_sc=False,
                                needs_layout_passes=False)


def _wid():
    return lax.axis_index("s") * 2 + lax.axis_index("c")


def _splat(x):
    return jnp.full((16,), x, jnp.int32)


def _elementwise(xg):
    e = jnp.exp(xg + jnp.float32(ACT_SHIFT))
    inv = 1.0 / (1.0 + e)
    q = jnp.maximum(inv, jnp.float32(1e-10))
    alpha = 1.0 - inv
    return alpha, q


def _init_prev_rid(rid_hbm, idx_v, prev_v, sem, base_reg, iota):
    idx_v[pl.ds(0, 16)] = jnp.maximum((base_reg + iota) * RLEN - 1, 0)
    idx_v[pl.ds(16, 16)] = (base_reg + 16 + iota) * RLEN - 1
    pltpu.async_copy(rid_hbm.at[idx_v], prev_v, sem).wait()
    pa = jnp.where((base_reg + iota) == 0, -1, prev_v[pl.ds(0, 16)])
    pb = prev_v[pl.ds(16, 16)]
    return pa, pb


def _issue_in(dens_hbm, rid_hbm, dbuf, rbuf, dsem, rsem, base_reg, b):
    cps = []
    for k in range(KREG):
        src = pl.ds((base_reg + k) * RLEN + b * BLK, BLK)
        dst = pl.ds(k * BLK, BLK)
        cps.append(pltpu.async_copy(dens_hbm.at[src], dbuf.at[dst], dsem))
        cps.append(pltpu.async_copy(rid_hbm.at[src], rbuf.at[dst], rsem))
    return cps


def _compose(cur, tail, has):
    return tail * jnp.where(has != 0, jnp.float32(1.0), cur)


def _ks_body(dens_hbm, rid_hbm,
             w_hbm, slabs_hbm, tails_hbm, has_hbm, hpos_hbm, hray_hbm,
             ttail_hbm, thas_hbm,
             d0, d1, r0, r1, w0, w1, seg_v,
             stfa, stfb, stia, stib, stt, sth,
             idx_v, prev_v,
             sp, sd0, sd1, sr0, sr1, sw0, sw1):
    w = _wid()
    base_reg = w * KREG
    iota = lax.iota(jnp.int32, 16)
    rowa = iota * BLK
    rowb = (16 + iota) * BLK
    preva, prevb = _init_prev_rid(rid_hbm, idx_v, prev_v, sp, base_reg, iota)

    dbuf, rbuf, wbuf = (d0, d1), (r0, r1), (w0, w1)
    dsem, rsem, wsem = (sd0, sd1), (sr0, sr1), (sw0, sw1)

    def issue(b):
        return _issue_in(dens_hbm, rid_hbm, dbuf[b % 2], rbuf[b % 2],
                         dsem[b % 2], rsem[b % 2], base_reg, b)

    pend = issue(0)
    ta = jnp.ones((16,), jnp.float32)
    tb = jnp.ones((16,), jnp.float32)

    nanv = jnp.full((16,), jnp.nan, jnp.float32)

    @plsc.parallel_loop(0, NRAYS // 16, unroll=8)
    def _init(i):
        seg_v[pl.ds(pl.multiple_of(i * 16, 16), 16)] = nanv

    zeros = jnp.zeros((16,), jnp.int32)
    rlens = jnp.full((16,), RLEN, jnp.int32)
    carry = (ta, preva, zeros, rlens, zeros, tb, prevb, zeros, rlens, zeros)
    wpend = [None, None]
    for b in range(NBLK):
        nxt = issue(b + 1) if b + 1 < NBLK else None
        for cp in pend:
            cp.wait()
        db = dbuf[b % 2]
        rb = rbuf[b % 2]
        wb = wbuf[b % 2]
        if wpend[b % 2] is not None:
            for cp in wpend[b % 2]:
                cp.wait()

        @plsc.parallel_loop(0, BLK, carry=carry, unroll=8)
        def _sweep2(j, c):
            ta, pa, ha, hpa, hra, tb, pb, hb, hpb, hrb = c
            col = _splat(j)
            jg = col + (b * BLK)
            xa = plsc.load_gather(db, [rowa + col])
            ra = plsc.load_gather(rb, [rowa + col])
            xb = plsc.load_gather(db, [rowb + col])
            rb_ = plsc.load_gather(rb, [rowb + col])
            bnda = ra != pa
            bndb = rb_ != pb
            newfa = bnda & (ha == 0)
            newfb = bndb & (hb == 0)
            plsc.store_scatter(seg_v, [pa], ta, mask=bnda & (pa >= 0))
            plsc.store_scatter(seg_v, [pb], tb, mask=bndb)
            ta = jnp.where(bnda, jnp.float32(1.0), ta)
            tb = jnp.where(bndb, jnp.float32(1.0), tb)
            alpha_a, qa = _elementwise(xa)
            alpha_b, qb = _elementwise(xb)
            plsc.store_scatter(wb, [rowa + col], alpha_a * ta)
            plsc.store_scatter(wb, [rowb + col], alpha_b * tb)
            ta = ta * qa
            tb = tb * qb
            hpa = jnp.where(newfa, jg, hpa)
            hpb = jnp.where(newfb, jg, hpb)
            hra = jnp.where(newfa, pa, hra)
            hrb = jnp.where(newfb, pb, hrb)
            ha = jnp.where(bnda, 1, ha)
            hb = jnp.where(bndb, 1, hb)
            return ta, ra, ha, hpa, hra, tb, rb_, hb, hpb, hrb
        carry = _sweep2
        cws = []
        for k in range(KREG):
            dst = pl.ds((base_reg + k) * RLEN + b * BLK, BLK)
            cws.append(pltpu.async_copy(
                wb.at[pl.ds(k * BLK, BLK)], w_hbm.at[dst], wsem[b % 2]))
        wpend[b % 2] = cws
        pend = nxt

    ta, pa, ha, hpa, hra, tb, pb, hb, hpb, hrb = carry
    last = (base_reg + 16 + iota) == (NREG - 1)
    plsc.store_scatter(seg_v, [pb], tb, mask=last)
    hrb = jnp.where(last & (hb == 0), pb, hrb)
    hb = jnp.where(last, 1, hb)

    stfa[...] = ta
    stfb[...] = tb
    pltpu.sync_copy(stfa, tails_hbm.at[pl.ds(base_reg, 16)])
    pltpu.sync_copy(stfb, tails_hbm.at[pl.ds(base_reg + 16, 16)])
    stia[...] = ha
    stib[...] = hb
    pltpu.sync_copy(stia, has_hbm.at[pl.ds(base_reg, 16)])
    pltpu.sync_copy(stib, has_hbm.at[pl.ds(base_reg + 16, 16)])
    stia[...] = hpa
    stib[...] = hpb
    pltpu.sync_copy(stia, hpos_hbm.at[pl.ds(base_reg, 16)])
    pltpu.sync_copy(stib, hpos_hbm.at[pl.ds(base_reg + 16, 16)])
    stia[...] = hra
    stib[...] = hrb
    pltpu.sync_copy(stia, hray_hbm.at[pl.ds(base_reg, 16)])
    pltpu.sync_copy(stib, hray_hbm.at[pl.ds(base_reg + 16, 16)])

    stfa[...] = ta
    stfb[...] = tb
    stia[...] = ha
    stib[...] = hb
    cur = jnp.ones((16,), jnp.float32)
    anyh = jnp.zeros((16,), jnp.int32)
    for k in range(KREG):
        tv, hv = (stfa, stia) if k < 16 else (stfb, stib)
        ti = plsc.load_gather(tv, [_splat(k % 16)])
        hi = plsc.load_gather(hv, [_splat(k % 16)])
        cur = _compose(cur, ti, hi)
        anyh = jnp.where(hi != 0, 1, anyh)
    stt[...] = cur
    sth[...] = anyh
    pltpu.sync_copy(stt, ttail_hbm.at[pl.ds(w * 16, 16)])
    pltpu.sync_copy(sth, thas_hbm.at[pl.ds(w * 16, 16)])

    for cws in wpend:
        if cws is not None:
            for cp in cws:
                cp.wait()
    pltpu.sync_copy(seg_v, slabs_hbm.at[pl.ds(w * NRAYS, NRAYS)])


def _k3_body(slabs_hbm, ainv_hbm, all_v, out_v, sem):
    w = _wid()
    nper = NRAYS // NTILES
    base_ray = w * nper
    cps = [pltpu.async_copy(slabs_hbm.at[pl.ds(tt * NRAYS + base_ray, nper)],
                            all_v.at[pl.ds(tt * nper, nper)], sem)
           for tt in range(NTILES)]
    for cp in cps:
        cp.wait()

    @plsc.parallel_loop(0, nper // 16, unroll=2)
    def _fold(i):
        off = pl.multiple_of(i * 16, 16)
        a = all_v[pl.ds(off, 16)]
        for tt in range(1, NTILES):
            b = all_v[pl.ds(tt * nper + off, 16)]
            a = jnp.where(a != a, b, a)
        out_v[pl.ds(off, 16)] = jnp.where(a != a, jnp.float32(1.0), a)

    pltpu.sync_copy(out_v, ainv_hbm.at[pl.ds(base_ray, nper)])


def _kf_body(w_hbm, slabs_hbm, tails_hbm, has_hbm, hpos_hbm, hray_hbm,
             ttail_hbm, thas_hbm, dum_hbm,
             mt_v, mh_v, mhp_v, mhr_v, tt_v, th_v, ins_v, wv,
             sidx_v, sval_v, st,
             sw, sg):
    w = _wid()
    base_reg = w * KREG
    iota = lax.iota(jnp.int32, 16)

    pltpu.sync_copy(ttail_hbm, tt_v)
    pltpu.sync_copy(thas_hbm, th_v)
    pltpu.sync_copy(tails_hbm.at[pl.ds(base_reg, KREG)], mt_v)
    pltpu.sync_copy(has_hbm.at[pl.ds(base_reg, KREG)], mh_v)
    pltpu.sync_copy(hpos_hbm.at[pl.ds(base_reg, KREG)], mhp_v)
    pltpu.sync_copy(hray_hbm.at[pl.ds(base_reg, KREG)], mhr_v)

    def cstep(i, cur):
        ti = plsc.load_gather(tt_v, [_splat(0) + i * 16])
        hi = plsc.load_gather(th_v, [_splat(0) + i * 16])
        return _compose(cur, ti, hi)

    cur = lax.fori_loop(0, w, cstep, jnp.ones((16,), jnp.float32))

    def estep(k, c):
        cur, ta, tb = c
        ta = jnp.where(iota == k, cur, ta)
        tb = jnp.where(iota == (k - 16), cur, tb)
        ti = plsc.load_gather(mt_v, [_splat(0) + k])
        hi = plsc.load_gather(mh_v, [_splat(0) + k])
        return _compose(cur, ti, hi), ta, tb

    ones16 = jnp.ones((16,), jnp.float32)
    _, ta, tb = lax.fori_loop(0, KREG, estep, (cur, ones16, ones16))
    ins_v[pl.ds(0, 16)] = ta
    ins_v[pl.ds(16, 16)] = tb

    def head_pass(c):
        cps = [pltpu.async_copy(
            w_hbm.at[pl.ds((base_reg + k) * RLEN + c * 512, 512)],
            wv.at[pl.ds(k * 512, 512)], sw) for k in range(KREG)]
        for cp in cps:
            cp.wait()

        @plsc.parallel_loop(0, KREG * 32, unroll=4)
        def _rs(i):
            k = lax.shift_right_logical(i, 5)
            pos = (i - k * 32) * 16 + iota + c * 512
            hp = plsc.load_gather(mhp_v, [_splat(0) + k])
            inn = plsc.load_gather(ins_v, [_splat(0) + k])
            off = pl.multiple_of(i * 16, 16)
            v = wv[pl.ds(off, 16)]
            wv[pl.ds(off, 16)] = jnp.where(pos < hp, v * inn, v)

        cps = [pltpu.async_copy(
            wv.at[pl.ds(k * 512, 512)],
            w_hbm.at[pl.ds((base_reg + k) * RLEN + c * 512, 512)], sw)
            for k in range(KREG)]
        for cp in cps:
            cp.wait()

    head_pass(0)
    hmax = jnp.max(jnp.maximum(mhp_v[pl.ds(0, 16)], mhp_v[pl.ds(16, 16)]))

    @pl.when(hmax > 512)
    def _slow():
        for c in range(1, RLEN // 512):
            head_pass(c)

    sidx_v[pl.ds(0, 16)] = w * NRAYS + mhr_v[pl.ds(0, 16)]
    sidx_v[pl.ds(16, 16)] = w * NRAYS + mhr_v[pl.ds(16, 16)]
    pltpu.async_copy(slabs_hbm.at[sidx_v], sval_v, sg).wait()
    ha = mh_v[pl.ds(0, 16)]
    hb = mh_v[pl.ds(16, 16)]
    onesf = jnp.ones((16,), jnp.float32)
    sval_v[pl.ds(0, 16)] = sval_v[pl.ds(0, 16)] * jnp.where(ha != 0, ta, onesf)
    sval_v[pl.ds(16, 16)] = sval_v[pl.ds(16, 16)] * jnp.where(hb != 0, tb, onesf)
    pltpu.async_copy(sval_v, slabs_hbm.at[sidx_v], sg).wait()

    st[...] = onesf
    pltpu.sync_copy(st, dum_hbm.at[pl.ds(w * 16, 16)])


_ks = pl.kernel(
    _ks_body,
    out_type=(jax.ShapeDtypeStruct((TOTAL,), jnp.float32),
              jax.ShapeDtypeStruct((NTILES * NRAYS,), jnp.float32),
              jax.ShapeDtypeStruct((NREG,), jnp.float32),
              jax.ShapeDtypeStruct((NREG,), jnp.int32),
              jax.ShapeDtypeStruct((NREG,), jnp.int32),
              jax.ShapeDtypeStruct((NREG,), jnp.int32),
              jax.ShapeDtypeStruct((NTILES * 16,), jnp.float32),
              jax.ShapeDtypeStruct((NTILES * 16,), jnp.int32)),
    mesh=_MESH,
    compiler_params=_CPARAMS,
    scratch_types=[
        pltpu.VMEM((KREG * BLK,), jnp.float32), pltpu.VMEM((KREG * BLK,), jnp.float32),
        pltpu.VMEM((KREG * BLK,), jnp.int32), pltpu.VMEM((KREG * BLK,), jnp.int32),
        pltpu.VMEM((KREG * BLK,), jnp.float32), pltpu.VMEM((KREG * BLK,), jnp.float32),
        pltpu.VMEM((NRAYS,), jnp.float32),
        pltpu.VMEM((16,), jnp.float32), pltpu.VMEM((16,), jnp.float32),
        pltpu.VMEM((16,), jnp.int32), pltpu.VMEM((16,), jnp.int32),
        pltpu.VMEM((16,), jnp.float32), pltpu.VMEM((16,), jnp.int32),
        pltpu.VMEM((32,), jnp.int32), pltpu.VMEM((32,), jnp.int32),
        pltpu.SemaphoreType.DMA, pltpu.SemaphoreType.DMA, pltpu.SemaphoreType.DMA,
        pltpu.SemaphoreType.DMA, pltpu.SemaphoreType.DMA, pltpu.SemaphoreType.DMA,
        pltpu.SemaphoreType.DMA,
    ],
)

_CPARAMS_F = pltpu.CompilerParams(use_tc_tiling_on_sc=False,
                                  needs_layout_passes=False,
                                  has_side_effects=True)

_kf = pl.kernel(
    _kf_body,
    out_type=jax.ShapeDtypeStruct((NTILES * 16,), jnp.float32),
    mesh=_MESH,
    compiler_params=_CPARAMS_F,
    scratch_types=[
        pltpu.VMEM((KREG,), jnp.float32), pltpu.VMEM((KREG,), jnp.int32),
        pltpu.VMEM((KREG,), jnp.int32), pltpu.VMEM((KREG,), jnp.int32),
        pltpu.VMEM((NTILES * 16,), jnp.float32), pltpu.VMEM((NTILES * 16,), jnp.int32),
        pltpu.VMEM((KREG,), jnp.float32),
        pltpu.VMEM((KREG * 512,), jnp.float32),
        pltpu.VMEM((KREG,), jnp.int32), pltpu.VMEM((KREG,), jnp.float32),
        pltpu.VMEM((16,), jnp.float32),
        pltpu.SemaphoreType.DMA, pltpu.SemaphoreType.DMA,
    ],
)

_k3 = pl.kernel(
    _k3_body,
    out_type=jax.ShapeDtypeStruct((NRAYS,), jnp.float32),
    mesh=_MESH,
    compiler_params=_CPARAMS,
    scratch_types=[
        pltpu.VMEM((NRAYS,), jnp.float32),
        pltpu.VMEM((NRAYS // NTILES,), jnp.float32),
        pltpu.SemaphoreType.DMA,
    ],
)


def kernel(density, ray_id, N):
    del N
    wloc, slabs, tails, has, hpos, hray, ttail, thas = _ks(density, ray_id)
    fdum = _kf(wloc, slabs, tails, has, hpos, hray, ttail, thas)
    weights, slabs2, _ = lax.optimization_barrier((wloc, slabs, fdum))
    alphainv = _k3(slabs2)
    return weights, alphainv
